# Initial kernel scaffold; baseline (speedup 1.0000x reference)
import jax
import jax.numpy as jnp
from jax.experimental import pallas as pl
from jax.experimental.pallas import tpu as pltpu

_EPS = 1e-06
_NUM_LABELS = 64
_NUM_ANCHORS = 9
_NUM_SEG = _NUM_LABELS * _NUM_ANCHORS


def _loss_kernel(pred_ref, gt_ref, an_ref, out_ref):
    i = pl.program_id(0)
    n_img = pl.num_programs(0)

    pred_t = pred_ref[0]
    gt = gt_ref[0]
    an = an_ref[0]

    c = an * _NUM_LABELS + gt
    seg_iota = jax.lax.broadcasted_iota(jnp.int32, (_NUM_SEG, 1), 0)
    memb = (c == seg_iota).astype(jnp.float32)

    counts = jnp.sum(memb, axis=1, keepdims=True)
    present = counts > 0.0
    present_f = present.astype(jnp.float32)
    safe_counts = jnp.where(present, counts, 1.0)

    tag_sums = jax.lax.dot_general(
        memb, pred_t, (((1,), (1,)), ((), ())),
        preferred_element_type=jnp.float32)
    tags = tag_sums / safe_counts
    nt = jnp.sqrt(jnp.sum(tags * tags, axis=1, keepdims=True))
    na = jnp.sqrt(jnp.sum(pred_t * pred_t, axis=0, keepdims=True))

    tag_g = jax.lax.dot_general(
        tags, memb, (((0,), (0,)), ((), ())),
        preferred_element_type=jnp.float32)
    nt_g = jax.lax.dot_general(
        nt, memb, (((0,), (0,)), ((), ())),
        preferred_element_type=jnp.float32)
    cnt_g = jax.lax.dot_general(
        safe_counts, memb, (((0,), (0,)), ((), ())),
        preferred_element_type=jnp.float32)

    dotraw = jnp.sum(tag_g * pred_t, axis=0, keepdims=True)
    cos_e = dotraw / jnp.maximum(nt_g * na, _EPS)
    v = (1.0 - cos_e) / cnt_g

    an_iota = jax.lax.broadcasted_iota(jnp.int32, (_NUM_ANCHORS, 1), 0)
    onehot_a = (an == an_iota).astype(jnp.float32)
    pull_num = jax.lax.dot_general(
        onehot_a, v, (((1,), (1,)), ((), ())),
        preferred_element_type=jnp.float32)
    els_a = jnp.sum(onehot_a, axis=1, keepdims=True)

    sel = (jax.lax.broadcasted_iota(jnp.int32, (_NUM_ANCHORS, _NUM_SEG), 1)
           // _NUM_LABELS
           == jax.lax.broadcasted_iota(
               jnp.int32, (_NUM_ANCHORS, _NUM_SEG), 0)).astype(jnp.float32)
    obj = jax.lax.dot_general(
        sel, present_f, (((1,), (0,)), ((), ())),
        preferred_element_type=jnp.float32)

    img_loss = jnp.float32(0.0)
    an_count = jnp.sum((els_a > 0.0).astype(jnp.float32))
    for a in range(_NUM_ANCHORS):
        lo = a * _NUM_LABELS
        ta = tags[lo:lo + _NUM_LABELS]
        nta = nt[lo:lo + _NUM_LABELS]
        pra = present_f[lo:lo + _NUM_LABELS]
        pair_dots = jax.lax.dot_general(
            ta, ta, (((1,), (1,)), ((), ())),
            preferred_element_type=jnp.float32)
        nt_outer = jax.lax.dot_general(
            nta, nta, (((1,), (1,)), ((), ())),
            preferred_element_type=jnp.float32)
        mask_outer = jax.lax.dot_general(
            pra, pra, (((1,), (1,)), ((), ())),
            preferred_element_type=jnp.float32)
        pair_cos = pair_dots / jnp.maximum(nt_outer, _EPS)
        obj_a = obj[a, 0]
        push_num = jnp.sum(mask_outer * (1.0 + pair_cos)) - obj_a * 2.0
        push_a = push_num / (((obj_a - 1.0) * obj_a + _EPS) * 2.0)
        pull_a = pull_num[a, 0] / (obj_a + _EPS)
        loss_a = jnp.where(obj_a <= 1.0, 0.0, pull_a + push_a)
        img_loss = img_loss + jnp.where(els_a[a, 0] > 0.0, loss_a, 0.0)

    img_loss = img_loss / an_count

    @pl.when(i == 0)
    def _():
        out_ref[0, 0] = 0.0

    out_ref[0, 0] += img_loss / n_img


def kernel(pred, gt_inds, anchor_inds):
    img_num, n, d = pred.shape
    pred_t = jnp.transpose(pred, (0, 2, 1))
    gt3 = gt_inds.astype(jnp.int32).reshape(img_num, 1, n)
    an3 = anchor_inds.astype(jnp.int32).reshape(img_num, 1, n)

    out = pl.pallas_call(
        _loss_kernel,
        grid=(img_num,),
        in_specs=[
            pl.BlockSpec((1, d, n), lambda i: (i, 0, 0)),
            pl.BlockSpec((1, 1, n), lambda i: (i, 0, 0)),
            pl.BlockSpec((1, 1, n), lambda i: (i, 0, 0)),
        ],
        out_specs=pl.BlockSpec((1, 1), lambda i: (0, 0)),
        out_shape=jax.ShapeDtypeStruct((1, 1), jnp.float32),
    )(pred_t, gt3, an3)
    return out[0, 0]


# speedup vs baseline: 11.6696x; 11.6696x over previous
import jax
import jax.numpy as jnp
from jax.experimental import pallas as pl
from jax.experimental.pallas import tpu as pltpu

_EPS = 1e-06
_NUM_LABELS = 64
_NUM_ANCHORS = 9
_NUM_SEG = _NUM_LABELS * _NUM_ANCHORS


def _loss_kernel(pred_ref, gt_ref, an_ref, out_ref):
    i = pl.program_id(0)
    n_img = pl.num_programs(0)

    pred_t = pred_ref[0]
    gt = gt_ref[0]
    an = an_ref[0]

    c = an * _NUM_LABELS + gt
    seg_iota = jax.lax.broadcasted_iota(jnp.int32, (_NUM_SEG, 1), 0)
    memb = (c == seg_iota).astype(jnp.float32)

    counts = jnp.sum(memb, axis=1, keepdims=True)
    present = counts > 0.0
    present_f = present.astype(jnp.float32)
    safe_counts = jnp.where(present, counts, 1.0)

    tag_sums = jax.lax.dot_general(
        memb, pred_t, (((1,), (1,)), ((), ())),
        preferred_element_type=jnp.float32)
    tags = tag_sums / safe_counts
    nt = jnp.sqrt(jnp.sum(tags * tags, axis=1, keepdims=True))
    na = jnp.sqrt(jnp.sum(pred_t * pred_t, axis=0, keepdims=True))

    tag_g = jax.lax.dot_general(
        tags, memb, (((0,), (0,)), ((), ())),
        preferred_element_type=jnp.float32)
    nt_g = jax.lax.dot_general(
        nt, memb, (((0,), (0,)), ((), ())),
        preferred_element_type=jnp.float32)
    cnt_g = jax.lax.dot_general(
        safe_counts, memb, (((0,), (0,)), ((), ())),
        preferred_element_type=jnp.float32)

    dotraw = jnp.sum(tag_g * pred_t, axis=0, keepdims=True)
    cos_e = dotraw / jnp.maximum(nt_g * na, _EPS)
    v = (1.0 - cos_e) / cnt_g

    an_iota = jax.lax.broadcasted_iota(jnp.int32, (_NUM_ANCHORS, 1), 0)
    onehot_a = (an == an_iota).astype(jnp.float32)
    pull_num = jax.lax.dot_general(
        onehot_a, v, (((1,), (1,)), ((), ())),
        preferred_element_type=jnp.float32)
    els_a = jnp.sum(onehot_a, axis=1, keepdims=True)

    sel = (jax.lax.broadcasted_iota(jnp.int32, (_NUM_ANCHORS, _NUM_SEG), 1)
           // _NUM_LABELS
           == jax.lax.broadcasted_iota(
               jnp.int32, (_NUM_ANCHORS, _NUM_SEG), 0)).astype(jnp.float32)
    obj = jax.lax.dot_general(
        sel, present_f, (((1,), (0,)), ((), ())),
        preferred_element_type=jnp.float32)

    img_loss = jnp.float32(0.0)
    an_count = jnp.sum((els_a > 0.0).astype(jnp.float32))
    for a in range(_NUM_ANCHORS):
        lo = a * _NUM_LABELS
        ta = tags[lo:lo + _NUM_LABELS]
        nta = nt[lo:lo + _NUM_LABELS]
        pra = present_f[lo:lo + _NUM_LABELS]
        pair_dots = jax.lax.dot_general(
            ta, ta, (((1,), (1,)), ((), ())),
            preferred_element_type=jnp.float32)
        nt_outer = jax.lax.dot_general(
            nta, nta, (((1,), (1,)), ((), ())),
            preferred_element_type=jnp.float32)
        mask_outer = jax.lax.dot_general(
            pra, pra, (((1,), (1,)), ((), ())),
            preferred_element_type=jnp.float32)
        pair_cos = pair_dots / jnp.maximum(nt_outer, _EPS)
        obj_a = obj[a, 0]
        push_num = jnp.sum(mask_outer * (1.0 + pair_cos)) - obj_a * 2.0
        push_a = push_num / (((obj_a - 1.0) * obj_a + _EPS) * 2.0)
        pull_a = pull_num[a, 0] / (obj_a + _EPS)
        loss_a = jnp.where(obj_a <= 1.0, 0.0, pull_a + push_a)
        img_loss = img_loss + jnp.where(els_a[a, 0] > 0.0, loss_a, 0.0)

    img_loss = img_loss / an_count

    @pl.when(i == 0)
    def _():
        out_ref[...] = jnp.zeros_like(out_ref)

    out_ref[...] = out_ref[...] + jnp.full((1, 1), img_loss / n_img,
                                           jnp.float32)


def kernel(pred, gt_inds, anchor_inds):
    img_num, n, d = pred.shape
    pred_t = jnp.transpose(pred, (0, 2, 1))
    gt3 = gt_inds.astype(jnp.int32).reshape(img_num, 1, n)
    an3 = anchor_inds.astype(jnp.int32).reshape(img_num, 1, n)

    out = pl.pallas_call(
        _loss_kernel,
        grid=(img_num,),
        in_specs=[
            pl.BlockSpec((1, d, n), lambda i: (i, 0, 0)),
            pl.BlockSpec((1, 1, n), lambda i: (i, 0, 0)),
            pl.BlockSpec((1, 1, n), lambda i: (i, 0, 0)),
        ],
        out_specs=pl.BlockSpec((1, 1), lambda i: (0, 0)),
        out_shape=jax.ShapeDtypeStruct((1, 1), jnp.float32),
    )(pred_t, gt3, an3)
    return out[0, 0]
